# baseline (speedup 1.0000x reference)
import functools

import jax
import jax.numpy as jnp
from jax import lax
from jax.experimental import pallas as pl
from jax.experimental.pallas import tpu as pltpu
from jax.experimental.pallas import tpu_sc as plsc

NC = 2
NS = 16
L = 16
NW = NC * NS


def kernel(triples, entity_re, entity_im, relation_re, relation_im):
    B = triples.shape[0]
    D = entity_re.shape[1]

    NB = 4
    CH = 32
    per_w = B // NW
    n_ch = per_w // CH
    U = 8

    mesh = plsc.VectorSubcoreMesh(core_axis_name="c", subcore_axis_name="s")

    vbuf = lambda: pltpu.VMEM((6 * CH, D), jnp.float32)

    @functools.partial(
        pl.kernel,
        mesh=mesh,
        compiler_params=pltpu.CompilerParams(needs_layout_passes=False),
        out_type=jax.ShapeDtypeStruct((B,), jnp.float32),
        scratch_types=[
            [pltpu.VMEM((per_w,), jnp.int32) for _ in range(3)],
            [vbuf() for _ in range(NB)],
            pltpu.VMEM((per_w,), jnp.float32),
            [pltpu.SemaphoreType.DMA for _ in range(NB)],
            pltpu.SemaphoreType.DMA,
        ],
    )
    def scmk(hidx_hbm, ridx_hbm, tidx_hbm, ere_hbm, eim_hbm, rre_hbm, rim_hbm,
             out_hbm, idx_s, buf_s, sc_v, sems, isem):
        wid = lax.axis_index("s") * NC + lax.axis_index("c")
        wbase = wid * per_w
        lanes = lax.iota(jnp.int32, L)
        tables = (ere_hbm, eim_hbm, rre_hbm, rim_hbm, ere_hbm, eim_hbm)

        ih_v, ir_v, it_v = idx_s
        icps = [
            pltpu.async_copy(hidx_hbm.at[pl.ds(wbase, per_w)], ih_v, isem),
            pltpu.async_copy(ridx_hbm.at[pl.ds(wbase, per_w)], ir_v, isem),
            pltpu.async_copy(tidx_hbm.at[pl.ds(wbase, per_w)], it_v, isem),
        ]
        for cp in icps:
            cp.wait()

        def issue(c, slot):
            off = c * CH
            ih = ih_v.at[pl.ds(off, CH)]
            ir = ir_v.at[pl.ds(off, CH)]
            it = it_v.at[pl.ds(off, CH)]
            srcs = (ih, ih, ir, ir, it, it)
            for k, (tab, src) in enumerate(zip(tables, srcs)):
                pltpu.async_copy(tab.at[src],
                                 buf_s[slot].at[pl.ds(k * CH, CH)],
                                 sems[slot])

        def drain(slot):
            pltpu.make_async_copy(ere_hbm.at[pl.ds(0, 6 * CH)], buf_s[slot],
                                  sems[slot]).wait()

        def compute(c, slot):
            buf = buf_s[slot]
            off = c * CH

            def group_body(g, carry2):
                rows = g * L + lanes
                zz = (jnp.zeros((L,), jnp.float32),
                      jnp.zeros((L,), jnp.float32))

                @plsc.parallel_loop(0, D, step=U, carry=zz)
                def dblk(d0, accs):
                    a0, a1 = accs
                    for u in range(U):
                        cols = (lanes * 9 + (d0 + u)) & (D - 1)
                        hre = plsc.load_gather(buf, [rows, cols])
                        him = plsc.load_gather(buf, [rows + CH, cols])
                        rre = plsc.load_gather(buf, [rows + 2 * CH, cols])
                        rim = plsc.load_gather(buf, [rows + 3 * CH, cols])
                        tre = plsc.load_gather(buf, [rows + 4 * CH, cols])
                        tim = plsc.load_gather(buf, [rows + 5 * CH, cols])
                        a0 = a0 + (hre * rre - him * rim) * tre
                        a1 = a1 + (hre * rim + him * rre) * tim
                    return (a0, a1)

                acc0, acc1 = dblk
                sc_v[pl.ds(off + g * L, L)] = acc0 + acc1
                return carry2

            lax.fori_loop(0, CH // L, group_body, 0)

        for b in range(NB):
            issue(b, b)

        def pair_body(p, carry):
            c0 = p * NB
            for b in range(NB):
                c = c0 + b
                drain(b)
                compute(c, b)

                @pl.when(c + NB < n_ch)
                def _():
                    issue(c + NB, b)
            return carry

        lax.fori_loop(0, n_ch // NB, pair_body, 0)
        pltpu.sync_copy(sc_v, out_hbm.at[pl.ds(wbase, per_w)])

    return scmk(triples[:, 0], triples[:, 1], triples[:, 2],
                entity_re, entity_im, relation_re, relation_im)


# speedup vs baseline: 1.0020x; 1.0020x over previous
import functools

import jax
import jax.numpy as jnp
from jax import lax
from jax.experimental import pallas as pl
from jax.experimental.pallas import tpu as pltpu
from jax.experimental.pallas import tpu_sc as plsc

NC = 2
NS = 16
L = 16
NW = NC * NS


def kernel(triples, entity_re, entity_im, relation_re, relation_im):
    B = triples.shape[0]
    D = entity_re.shape[1]

    NB = 4
    CH = 32
    per_w = B // NW
    n_ch = per_w // CH
    U = 8

    mesh = plsc.VectorSubcoreMesh(core_axis_name="c", subcore_axis_name="s")

    vbuf = lambda: pltpu.VMEM((CH, D), jnp.float32)

    @functools.partial(
        pl.kernel,
        mesh=mesh,
        compiler_params=pltpu.CompilerParams(needs_layout_passes=False),
        out_type=jax.ShapeDtypeStruct((B,), jnp.float32),
        scratch_types=[
            [pltpu.VMEM((per_w,), jnp.int32) for _ in range(3)],
            [[vbuf() for _ in range(6)] for _ in range(NB)],
            pltpu.VMEM((per_w,), jnp.float32),
            [pltpu.SemaphoreType.DMA for _ in range(NB)],
            pltpu.SemaphoreType.DMA,
        ],
    )
    def scmk(hidx_hbm, ridx_hbm, tidx_hbm, ere_hbm, eim_hbm, rre_hbm, rim_hbm,
             out_hbm, idx_s, buf_s, sc_v, sems, isem):
        wid = lax.axis_index("s") * NC + lax.axis_index("c")
        wbase = wid * per_w
        lanes = lax.iota(jnp.int32, L)
        tables = (ere_hbm, eim_hbm, rre_hbm, rim_hbm, ere_hbm, eim_hbm)

        ih_v, ir_v, it_v = idx_s
        icps = [
            pltpu.async_copy(hidx_hbm.at[pl.ds(wbase, per_w)], ih_v, isem),
            pltpu.async_copy(ridx_hbm.at[pl.ds(wbase, per_w)], ir_v, isem),
            pltpu.async_copy(tidx_hbm.at[pl.ds(wbase, per_w)], it_v, isem),
        ]
        for cp in icps:
            cp.wait()

        def issue(c, slot):
            off = c * CH
            ih = ih_v.at[pl.ds(off, CH)]
            ir = ir_v.at[pl.ds(off, CH)]
            it = it_v.at[pl.ds(off, CH)]
            srcs = (ih, ih, ir, ir, it, it)
            for tab, src, buf in zip(tables, srcs, buf_s[slot]):
                pltpu.async_copy(tab.at[src], buf, sems[slot])

        def drain(slot):
            for tab, buf in zip(tables, buf_s[slot]):
                pltpu.make_async_copy(tab.at[ih_v.at[pl.ds(0, CH)]], buf,
                                      sems[slot]).wait()

        def compute(c, slot):
            hre_v, him_v, rre_v, rim_v, tre_v, tim_v = buf_s[slot]
            off = c * CH

            def group_body(g, carry2):
                rows = g * L + lanes
                zz = (jnp.zeros((L,), jnp.float32),
                      jnp.zeros((L,), jnp.float32))

                @plsc.parallel_loop(0, D, step=U, carry=zz)
                def dblk(d0, accs):
                    a0, a1 = accs
                    for u in range(U):
                        cols = (lanes * 9 + (d0 + u)) & (D - 1)
                        idx = [rows, cols]
                        hre = plsc.load_gather(hre_v, idx)
                        him = plsc.load_gather(him_v, idx)
                        rre = plsc.load_gather(rre_v, idx)
                        rim = plsc.load_gather(rim_v, idx)
                        tre = plsc.load_gather(tre_v, idx)
                        tim = plsc.load_gather(tim_v, idx)
                        a0 = a0 + (hre * rre - him * rim) * tre
                        a1 = a1 + (hre * rim + him * rre) * tim
                    return (a0, a1)

                acc0, acc1 = dblk
                sc_v[pl.ds(off + g * L, L)] = acc0 + acc1
                return carry2

            lax.fori_loop(0, CH // L, group_body, 0)

        for b in range(NB):
            issue(b, b)

        def pair_body(p, carry):
            c0 = p * NB
            for b in range(NB):
                c = c0 + b
                drain(b)
                compute(c, b)

                @pl.when(c + NB < n_ch)
                def _():
                    issue(c + NB, b)
            return carry

        lax.fori_loop(0, n_ch // NB, pair_body, 0)
        pltpu.sync_copy(sc_v, out_hbm.at[pl.ds(wbase, per_w)])

    return scmk(triples[:, 0], triples[:, 1], triples[:, 2],
                entity_re, entity_im, relation_re, relation_im)
